# baseline (speedup 1.0000x reference)
import math

import jax
import jax.numpy as jnp
import numpy as np
from jax.experimental import pallas as pl
from jax.experimental.pallas import tpu as pltpu

_LB = 512
_LAMBDA_DRIFT = 0.3


def _pass1_body(x_ref, snap_ref, wd_ref, wx_ref, wp_ref, g12_ref, g31_ref,
                wo_ref, bd_ref, bu_ref, bg_ref, bo_ref,
                raw_ref, ms_ref, qd_ref):
    li = pl.program_id(1)
    T = snap_ref.shape[1]
    snap = snap_ref[0]
    xb = x_ref[0]
    xp = snap[T - 1]
    delta = xb - xp
    f32 = jnp.float32
    df = jnp.dot(delta, wd_ref[...], preferred_element_type=f32) + bd_ref[...]
    u = (jnp.dot(xb, wx_ref[...], preferred_element_type=f32)
         - jnp.dot(xp, wp_ref[...], preferred_element_type=f32) + bu_ref[...])
    g = jax.nn.sigmoid(jnp.dot(xb, g12_ref[...], preferred_element_type=f32)
                       + jnp.dot(xp, g31_ref[...], preferred_element_type=f32)
                       + bg_ref[...])
    core = g * df + (1.0 - g) * u
    raw = jnp.dot(core, wo_ref[...], preferred_element_type=f32) + bo_ref[...]
    raw_ref[0] = raw

    msp = jnp.sum(snap, axis=1)
    qp = jnp.sum(xb + raw, axis=0, keepdims=True)
    dp = jnp.sum(delta, axis=0, keepdims=True)
    qdp = jnp.concatenate([qp, dp], axis=0)

    @pl.when(li == 0)
    def _():
        ms_ref[0] = msp
        qd_ref[0] = qdp

    @pl.when(li > 0)
    def _():
        ms_ref[0] += msp
        qd_ref[0] += qdp


def _scores_body(ms_ref, qd_ref, pe_ref, seqw_ref, seqb_ref, qw_ref, qb_ref,
                 memw_ref, memb_ref, curdw_ref, curdb_ref, memdw_ref,
                 memdb_ref, f2_ref, fuseb_ref, attn_ref, pw_ref):
    B, T, D = ms_ref.shape
    L = 1024.0
    f32 = jnp.float32
    ms_mean = ms_ref[...] * (1.0 / L)
    pep = jnp.dot(pe_ref[...], seqw_ref[...],
                  preferred_element_type=f32) + seqb_ref[...]
    m = ms_mean + pep[None]
    memg = (jnp.dot(m.reshape(B * T, D), memw_ref[...],
                    preferred_element_type=f32)
            + memb_ref[...]).reshape(B, T, D)
    qg = jnp.dot(qd_ref[:, 0, :] * (1.0 / L), qw_ref[...],
                 preferred_element_type=f32) + qb_ref[...]
    content = jnp.sum(qg[:, None, :] * memg, axis=-1) / math.sqrt(D)
    cur = jnp.dot(qd_ref[:, 1, :] * (1.0 / L), curdw_ref[...],
                  preferred_element_type=f32) + curdb_ref[...]
    mprev = jnp.concatenate(
        [jnp.zeros((B, 1, D), f32), m[:, :-1]], axis=1)
    md = (m - mprev).reshape(B * T, D)
    memd = (jnp.dot(md, memdw_ref[...], preferred_element_type=f32)
            + memdb_ref[...]).reshape(B, T, D)
    drift = -jnp.mean((cur[:, None, :] - memd) ** 2, axis=-1)
    logits = content + _LAMBDA_DRIFT * drift
    mx = jnp.max(logits, axis=1, keepdims=True)
    e = jnp.exp(logits - mx)
    attn = e / jnp.sum(e, axis=1, keepdims=True)
    attn_ref[...] = attn
    pe_w = jnp.sum(attn[:, :, None] * pep[None], axis=1)
    pw2 = jnp.dot(pe_w, f2_ref[...],
                  preferred_element_type=f32) + fuseb_ref[...]
    pw_ref[...] = jnp.concatenate(
        [pe_w[:, None, :], pw2[:, None, :]], axis=1)


def _pass2_body(x_ref, snap_ref, raw_ref, attn_ref, pw_ref, f1_ref, f2_ref,
                y_ref):
    b = pl.program_id(0)
    T = snap_ref.shape[1]
    snap = snap_ref[0]
    xb = x_ref[0]
    raw = raw_ref[0]
    acc = snap[0] * attn_ref[b, 0]
    for t in range(1, T):
        acc = acc + snap[t] * attn_ref[b, t]
    pwv = pw_ref[0]
    enh = acc + pwv[0:1]
    f32 = jnp.float32
    fpre = (jnp.dot(xb, f1_ref[...], preferred_element_type=f32)
            + jnp.dot(acc, f2_ref[...], preferred_element_type=f32)
            + pwv[1:2])
    fg = jax.nn.sigmoid(fpre)
    y_ref[0] = xb + raw + fg * enh


def _sinusoid_np(T, d):
    half = d // 2
    pos = np.arange(1, T + 1, dtype=np.float64)
    div = np.exp(-math.log(10000.0) * (2.0 * np.arange(half) / d))
    ang = pos[:, None] * div[None, :]
    pe = np.stack([np.sin(ang), np.cos(ang)], axis=-1)
    return jnp.asarray(pe.reshape(T, d), dtype=jnp.float32)


def kernel(x, memory_snapshot, delta_W, delta_b, xproj_W, xproj_b, phys_W,
           phys_b, gate_W, gate_b, outp_W, outp_b, seq_W, seq_b, q_W, q_b,
           mem_W, mem_b, curd_W, curd_b, memd_W, memd_b, fuse_W, fuse_b):
    B, T, L, D = memory_snapshot.shape
    Lb = _LB
    nL = L // Lb

    g1 = gate_W[0:D]
    g12 = g1 + gate_W[D:2 * D]
    g31 = gate_W[2 * D:3 * D] - g1
    f1 = fuse_W[0:D]
    f2 = fuse_W[D:2 * D]
    bu = (xproj_b - phys_b).reshape(1, D)
    r1 = lambda v: v.reshape(1, D)

    wspec = pl.BlockSpec((D, D), lambda b_, l_: (0, 0))
    bspec = pl.BlockSpec((1, D), lambda b_, l_: (0, 0))
    raw, ms_sum, qd_sum = pl.pallas_call(
        _pass1_body,
        grid=(B, nL),
        in_specs=[
            pl.BlockSpec((1, Lb, D), lambda b_, l_: (b_, l_, 0)),
            pl.BlockSpec((1, T, Lb, D), lambda b_, l_: (b_, 0, l_, 0)),
            wspec, wspec, wspec, wspec, wspec, wspec,
            bspec, bspec, bspec, bspec,
        ],
        out_specs=[
            pl.BlockSpec((1, Lb, D), lambda b_, l_: (b_, l_, 0)),
            pl.BlockSpec((1, T, D), lambda b_, l_: (b_, 0, 0)),
            pl.BlockSpec((1, 2, D), lambda b_, l_: (b_, 0, 0)),
        ],
        out_shape=[
            jax.ShapeDtypeStruct((B, L, D), jnp.float32),
            jax.ShapeDtypeStruct((B, T, D), jnp.float32),
            jax.ShapeDtypeStruct((B, 2, D), jnp.float32),
        ],
        compiler_params=pltpu.CompilerParams(
            dimension_semantics=("parallel", "arbitrary"),
            vmem_limit_bytes=56 * 1024 * 1024,
        ),
        name="dalm_pass1",
    )(x, memory_snapshot, delta_W, xproj_W, phys_W, g12, g31, outp_W,
      r1(delta_b), bu, r1(gate_b), r1(outp_b))

    pe = _sinusoid_np(T, D)
    attn, pw = pl.pallas_call(
        _scores_body,
        out_shape=[
            jax.ShapeDtypeStruct((B, T), jnp.float32),
            jax.ShapeDtypeStruct((B, 2, D), jnp.float32),
        ],
        name="dalm_scores",
    )(ms_sum, qd_sum, pe, seq_W, r1(seq_b), q_W, r1(q_b), mem_W, r1(mem_b),
      curd_W, r1(curd_b), memd_W, r1(memd_b), f2, r1(fuse_b))

    y = pl.pallas_call(
        _pass2_body,
        grid=(B, nL),
        in_specs=[
            pl.BlockSpec((1, Lb, D), lambda b_, l_: (b_, l_, 0)),
            pl.BlockSpec((1, T, Lb, D), lambda b_, l_: (b_, 0, l_, 0)),
            pl.BlockSpec((1, Lb, D), lambda b_, l_: (b_, l_, 0)),
            pl.BlockSpec(memory_space=pltpu.SMEM),
            pl.BlockSpec((1, 2, D), lambda b_, l_: (b_, 0, 0)),
            pl.BlockSpec((D, D), lambda b_, l_: (0, 0)),
            pl.BlockSpec((D, D), lambda b_, l_: (0, 0)),
        ],
        out_specs=pl.BlockSpec((1, Lb, D), lambda b_, l_: (b_, l_, 0)),
        out_shape=jax.ShapeDtypeStruct((B, L, D), jnp.float32),
        compiler_params=pltpu.CompilerParams(
            dimension_semantics=("parallel", "arbitrary"),
            vmem_limit_bytes=56 * 1024 * 1024,
        ),
        name="dalm_pass2",
    )(x, memory_snapshot, raw, attn, pw, f1, f2)
    return y


# speedup vs baseline: 1.0271x; 1.0271x over previous
import math

import jax
import jax.numpy as jnp
import numpy as np
from jax.experimental import pallas as pl
from jax.experimental.pallas import tpu as pltpu

_LB = 256
_LAMBDA_DRIFT = 0.3


def _fused_body(x_ref, snap_ref, pe_ref,
                wd_ref, wx_ref, wp_ref, g12_ref, g31_ref, wo_ref,
                seqw_ref, qw_ref, memw_ref, curdw_ref, memdw_ref,
                f1_ref, f2_ref,
                bd_ref, bu_ref, bg_ref, bo_ref, seqb_ref, qb_ref, memb_ref,
                curdb_ref, memdb_ref, fuseb_ref,
                y_ref,
                raw_scr, ms_scr, qd_scr, attn_scr, attn_smem, pw_scr, sem):
    p = pl.program_id(1)
    li = pl.program_id(2)
    T, Lb = snap_ref.shape[1], snap_ref.shape[2]
    D = snap_ref.shape[3]
    L = float(Lb) * pl.num_programs(2)
    f32 = jnp.float32
    snap = snap_ref[0]
    xb = x_ref[0]

    @pl.when(p == 0)
    def _phase0():
        xp = snap[T - 1]
        delta = xb - xp
        df = (jnp.dot(delta, wd_ref[...], preferred_element_type=f32)
              + bd_ref[...])
        u = (jnp.dot(xb, wx_ref[...], preferred_element_type=f32)
             - jnp.dot(xp, wp_ref[...], preferred_element_type=f32)
             + bu_ref[...])
        g = jax.nn.sigmoid(
            jnp.dot(xb, g12_ref[...], preferred_element_type=f32)
            + jnp.dot(xp, g31_ref[...], preferred_element_type=f32)
            + bg_ref[...])
        core = g * df + (1.0 - g) * u
        raw = (jnp.dot(core, wo_ref[...], preferred_element_type=f32)
               + bo_ref[...])
        raw_scr[li] = raw

        msp = jnp.sum(snap, axis=1)
        qp = jnp.sum(xb + raw, axis=0, keepdims=True)
        dp = jnp.sum(delta, axis=0, keepdims=True)
        qdp = jnp.concatenate([qp, dp], axis=0)

        @pl.when(li == 0)
        def _():
            ms_scr[...] = msp
            qd_scr[...] = qdp

        @pl.when(li > 0)
        def _():
            ms_scr[...] += msp
            qd_scr[...] += qdp

    @pl.when(jnp.logical_and(p == 1, li == 0))
    def _scores():
        ms_mean = ms_scr[...] * (1.0 / L)
        pep = (jnp.dot(pe_ref[...], seqw_ref[...],
                       preferred_element_type=f32) + seqb_ref[...])
        m = ms_mean + pep
        memg = (jnp.dot(m, memw_ref[...], preferred_element_type=f32)
                + memb_ref[...])
        qg = (jnp.dot(qd_scr[0:1] * (1.0 / L), qw_ref[...],
                      preferred_element_type=f32) + qb_ref[...])
        content = jnp.sum(qg * memg, axis=-1, keepdims=True) / math.sqrt(D)
        cur = (jnp.dot(qd_scr[1:2] * (1.0 / L), curdw_ref[...],
                       preferred_element_type=f32) + curdb_ref[...])
        mprev = jnp.concatenate(
            [jnp.zeros((1, D), f32), m[:-1]], axis=0)
        memd = (jnp.dot(m - mprev, memdw_ref[...],
                        preferred_element_type=f32) + memdb_ref[...])
        drift = -jnp.mean((cur - memd) ** 2, axis=-1, keepdims=True)
        logits = content + _LAMBDA_DRIFT * drift
        mx = jnp.max(logits, axis=0, keepdims=True)
        e = jnp.exp(logits - mx)
        attn = e / jnp.sum(e, axis=0, keepdims=True)
        attn_scr[...] = attn
        pe_w = jnp.sum(attn * pep, axis=0, keepdims=True)
        pw2 = (jnp.dot(pe_w, f2_ref[...], preferred_element_type=f32)
               + fuseb_ref[...])
        pw_scr[...] = jnp.concatenate([pe_w, pw2], axis=0)
        cp = pltpu.make_async_copy(attn_scr, attn_smem, sem)
        cp.start()
        cp.wait()

    @pl.when(p == 1)
    def _phase1():
        acc = snap[0] * attn_smem[0, 0]
        for t in range(1, T):
            acc = acc + snap[t] * attn_smem[t, 0]
        raw = raw_scr[li]
        pwv = pw_scr[...]
        enh = acc + pwv[0:1]
        fpre = (jnp.dot(xb, f1_ref[...], preferred_element_type=f32)
                + jnp.dot(acc, f2_ref[...], preferred_element_type=f32)
                + pwv[1:2])
        fg = jax.nn.sigmoid(fpre)
        y_ref[0] = xb + raw + fg * enh


def _sinusoid_np(T, d):
    half = d // 2
    pos = np.arange(1, T + 1, dtype=np.float64)
    div = np.exp(-math.log(10000.0) * (2.0 * np.arange(half) / d))
    ang = pos[:, None] * div[None, :]
    pe = np.stack([np.sin(ang), np.cos(ang)], axis=-1)
    return jnp.asarray(pe.reshape(T, d), dtype=jnp.float32)


def kernel(x, memory_snapshot, delta_W, delta_b, xproj_W, xproj_b, phys_W,
           phys_b, gate_W, gate_b, outp_W, outp_b, seq_W, seq_b, q_W, q_b,
           mem_W, mem_b, curd_W, curd_b, memd_W, memd_b, fuse_W, fuse_b):
    B, T, L, D = memory_snapshot.shape
    Lb = _LB
    nL = L // Lb

    g1 = gate_W[0:D]
    g12 = g1 + gate_W[D:2 * D]
    g31 = gate_W[2 * D:3 * D] - g1
    f1 = fuse_W[0:D]
    f2 = fuse_W[D:2 * D]
    bu = (xproj_b - phys_b).reshape(1, D)
    r1 = lambda v: v.reshape(1, D)
    pe = _sinusoid_np(T, D)

    wspec = pl.BlockSpec((D, D), lambda b_, p_, l_: (0, 0))
    bspec = pl.BlockSpec((1, D), lambda b_, p_, l_: (0, 0))
    y = pl.pallas_call(
        _fused_body,
        grid=(B, 2, nL),
        in_specs=[
            pl.BlockSpec((1, Lb, D), lambda b_, p_, l_: (b_, l_, 0)),
            pl.BlockSpec((1, T, Lb, D), lambda b_, p_, l_: (b_, 0, l_, 0)),
            pl.BlockSpec((T, D), lambda b_, p_, l_: (0, 0)),
            wspec, wspec, wspec, wspec, wspec, wspec,
            wspec, wspec, wspec, wspec, wspec,
            wspec, wspec,
            bspec, bspec, bspec, bspec, bspec, bspec, bspec, bspec, bspec,
            bspec,
        ],
        out_specs=pl.BlockSpec((1, Lb, D),
                               lambda b_, p_, l_: (b_, l_ * p_, 0)),
        out_shape=jax.ShapeDtypeStruct((B, L, D), jnp.float32),
        scratch_shapes=[
            pltpu.VMEM((nL, Lb, D), jnp.float32),
            pltpu.VMEM((T, D), jnp.float32),
            pltpu.VMEM((2, D), jnp.float32),
            pltpu.VMEM((T, 1), jnp.float32),
            pltpu.SMEM((T, 1), jnp.float32),
            pltpu.VMEM((2, D), jnp.float32),
            pltpu.SemaphoreType.DMA,
        ],
        compiler_params=pltpu.CompilerParams(
            dimension_semantics=("parallel", "arbitrary", "arbitrary"),
            vmem_limit_bytes=56 * 1024 * 1024,
        ),
        name="dalm_fused",
    )(x, memory_snapshot, pe,
      delta_W, xproj_W, phys_W, g12, g31, outp_W,
      seq_W, q_W, mem_W, curd_W, memd_W, f1, f2,
      r1(delta_b), bu, r1(gate_b), r1(outp_b), r1(seq_b), r1(q_b),
      r1(mem_b), r1(curd_b), r1(memd_b), r1(fuse_b))
    return y


# speedup vs baseline: 1.2211x; 1.1889x over previous
import math

import jax
import jax.numpy as jnp
import numpy as np
from jax.experimental import pallas as pl
from jax.experimental.pallas import tpu as pltpu

_LB = 256
_LAMBDA_DRIFT = 0.3
_WD, _WX, _WP, _G12, _G31, _WO, _SEQ, _Q, _MEM, _CURD, _MEMD, _F1, _F2 = (
    range(13))


def _fused_body(x_ref, snap_ref, pe_ref,
                wd_ref, wx_ref, wp_ref, g12_ref, g31_ref, wo_ref,
                seqw_ref, qw_ref, memw_ref, curdw_ref, memdw_ref,
                f1_ref, f2_ref,
                bd_ref, bu_ref, bg_ref, bo_ref, seqb_ref, qb_ref, memb_ref,
                curdb_ref, memdb_ref, fuseb_ref,
                y_ref,
                w_scr, snap_bf, x_scr, raw_scr, ms_scr, qd_scr, attn_scr,
                attn_smem, pw_scr, sem, wsem):
    b = pl.program_id(0)
    p = pl.program_id(1)
    li = pl.program_id(2)
    T, Lb = snap_ref.shape[1], snap_ref.shape[2]
    D = snap_ref.shape[3]
    nL = pl.num_programs(2)
    L = float(Lb) * nL
    f32 = jnp.float32

    @pl.when(jnp.logical_and(b == 0, jnp.logical_and(p == 0, li == 0)))
    def _load_weights():
        hbm = [wd_ref, wx_ref, wp_ref, g12_ref, g31_ref, wo_ref, seqw_ref,
               qw_ref, memw_ref, curdw_ref, memdw_ref, f1_ref, f2_ref]
        cps = [pltpu.make_async_copy(r, w_scr.at[i], wsem)
               for i, r in enumerate(hbm)]
        for cp in cps:
            cp.start()
        for cp in cps:
            cp.wait()

    @pl.when(p == 0)
    def _phase0():
        snap = snap_ref[0]
        xb = x_ref[0]
        xp = snap[T - 1]
        delta = xb - xp
        df = (jnp.dot(delta, w_scr[_WD], preferred_element_type=f32)
              + bd_ref[...])
        u = (jnp.dot(xb, w_scr[_WX], preferred_element_type=f32)
             - jnp.dot(xp, w_scr[_WP], preferred_element_type=f32)
             + bu_ref[...])
        g = jax.nn.sigmoid(
            jnp.dot(xb, w_scr[_G12], preferred_element_type=f32)
            + jnp.dot(xp, w_scr[_G31], preferred_element_type=f32)
            + bg_ref[...])
        core = g * df + (1.0 - g) * u
        raw = (jnp.dot(core, w_scr[_WO], preferred_element_type=f32)
               + bo_ref[...])
        raw_scr[li] = raw
        x_scr[li] = xb
        for t in range(T):
            snap_bf[li, t] = snap[t].astype(jnp.bfloat16)

        msp = jnp.sum(snap, axis=1)
        qp = jnp.sum(xb + raw, axis=0, keepdims=True)
        dp = jnp.sum(delta, axis=0, keepdims=True)
        qdp = jnp.concatenate([qp, dp], axis=0)

        @pl.when(li == 0)
        def _():
            ms_scr[...] = msp
            qd_scr[...] = qdp

        @pl.when(li > 0)
        def _():
            ms_scr[...] += msp
            qd_scr[...] += qdp

    @pl.when(jnp.logical_and(p == 1, li == 0))
    def _scores():
        ms_mean = ms_scr[...] * (1.0 / L)
        pep = (jnp.dot(pe_ref[...], w_scr[_SEQ],
                       preferred_element_type=f32) + seqb_ref[...])
        m = ms_mean + pep
        memg = (jnp.dot(m, w_scr[_MEM], preferred_element_type=f32)
                + memb_ref[...])
        qg = (jnp.dot(qd_scr[0:1] * (1.0 / L), w_scr[_Q],
                      preferred_element_type=f32) + qb_ref[...])
        content = jnp.sum(qg * memg, axis=-1, keepdims=True) / math.sqrt(D)
        cur = (jnp.dot(qd_scr[1:2] * (1.0 / L), w_scr[_CURD],
                       preferred_element_type=f32) + curdb_ref[...])
        mprev = jnp.concatenate(
            [jnp.zeros((1, D), f32), m[:-1]], axis=0)
        memd = (jnp.dot(m - mprev, w_scr[_MEMD],
                        preferred_element_type=f32) + memdb_ref[...])
        drift = -jnp.mean((cur - memd) ** 2, axis=-1, keepdims=True)
        logits = content + _LAMBDA_DRIFT * drift
        mx = jnp.max(logits, axis=0, keepdims=True)
        e = jnp.exp(logits - mx)
        attn = e / jnp.sum(e, axis=0, keepdims=True)
        attn_scr[...] = attn
        pe_w = jnp.sum(attn * pep, axis=0, keepdims=True)
        pw2 = (jnp.dot(pe_w, w_scr[_F2], preferred_element_type=f32)
               + fuseb_ref[...])
        pw_scr[...] = jnp.concatenate([pe_w, pw2], axis=0)
        cp = pltpu.make_async_copy(attn_scr, attn_smem, sem)
        cp.start()
        cp.wait()

    @pl.when(p == 1)
    def _phase1():
        acc = snap_bf[li, 0].astype(f32) * attn_smem[0, 0]
        for t in range(1, T):
            acc = acc + snap_bf[li, t].astype(f32) * attn_smem[t, 0]
        xb = x_scr[li]
        raw = raw_scr[li]
        pwv = pw_scr[...]
        enh = acc + pwv[0:1]
        fpre = (jnp.dot(xb, w_scr[_F1], preferred_element_type=f32)
                + jnp.dot(acc, w_scr[_F2], preferred_element_type=f32)
                + pwv[1:2])
        fg = jax.nn.sigmoid(fpre)
        y_ref[0] = xb + raw + fg * enh


def _sinusoid_np(T, d):
    half = d // 2
    pos = np.arange(1, T + 1, dtype=np.float64)
    div = np.exp(-math.log(10000.0) * (2.0 * np.arange(half) / d))
    ang = pos[:, None] * div[None, :]
    pe = np.stack([np.sin(ang), np.cos(ang)], axis=-1)
    return jnp.asarray(pe.reshape(T, d), dtype=jnp.float32)


def kernel(x, memory_snapshot, delta_W, delta_b, xproj_W, xproj_b, phys_W,
           phys_b, gate_W, gate_b, outp_W, outp_b, seq_W, seq_b, q_W, q_b,
           mem_W, mem_b, curd_W, curd_b, memd_W, memd_b, fuse_W, fuse_b):
    B, T, L, D = memory_snapshot.shape
    Lb = _LB
    nL = L // Lb

    g1 = gate_W[0:D]
    g12 = g1 + gate_W[D:2 * D]
    g31 = gate_W[2 * D:3 * D] - g1
    f1 = fuse_W[0:D]
    f2 = fuse_W[D:2 * D]
    bu = (xproj_b - phys_b).reshape(1, D)
    r1 = lambda v: v.reshape(1, D)
    pe = _sinusoid_np(T, D)

    anyspec = pl.BlockSpec(memory_space=pl.ANY)
    bspec = pl.BlockSpec((1, D), lambda b_, p_, l_: (0, 0))
    pinned = lambda b_, p_, l_: (b_, l_ * (1 - p_) + (nL - 1) * p_, 0)
    y = pl.pallas_call(
        _fused_body,
        grid=(B, 2, nL),
        in_specs=[
            pl.BlockSpec((1, Lb, D), pinned),
            pl.BlockSpec((1, T, Lb, D),
                         lambda b_, p_, l_:
                         (b_, 0, l_ * (1 - p_) + (nL - 1) * p_, 0)),
            pl.BlockSpec((T, D), lambda b_, p_, l_: (0, 0)),
            anyspec, anyspec, anyspec, anyspec, anyspec, anyspec,
            anyspec, anyspec, anyspec, anyspec, anyspec,
            anyspec, anyspec,
            bspec, bspec, bspec, bspec, bspec, bspec, bspec, bspec, bspec,
            bspec,
        ],
        out_specs=pl.BlockSpec((1, Lb, D),
                               lambda b_, p_, l_: (b_, l_ * p_, 0)),
        out_shape=jax.ShapeDtypeStruct((B, L, D), jnp.float32),
        scratch_shapes=[
            pltpu.VMEM((13, D, D), jnp.float32),
            pltpu.VMEM((nL, T, Lb, D), jnp.bfloat16),
            pltpu.VMEM((nL, Lb, D), jnp.float32),
            pltpu.VMEM((nL, Lb, D), jnp.float32),
            pltpu.VMEM((T, D), jnp.float32),
            pltpu.VMEM((2, D), jnp.float32),
            pltpu.VMEM((T, 1), jnp.float32),
            pltpu.SMEM((T, 1), jnp.float32),
            pltpu.VMEM((2, D), jnp.float32),
            pltpu.SemaphoreType.DMA,
            pltpu.SemaphoreType.DMA,
        ],
        compiler_params=pltpu.CompilerParams(
            dimension_semantics=("parallel", "arbitrary", "arbitrary"),
            vmem_limit_bytes=56 * 1024 * 1024,
        ),
        name="dalm_fused",
    )(x, memory_snapshot, pe,
      delta_W, xproj_W, phys_W, g12, g31, outp_W,
      seq_W, q_W, mem_W, curd_W, memd_W, f1, f2,
      r1(delta_b), bu, r1(gate_b), r1(outp_b), r1(seq_b), r1(q_b),
      r1(mem_b), r1(curd_b), r1(memd_b), r1(fuse_b))
    return y


# speedup vs baseline: 1.2953x; 1.0608x over previous
import math

import jax
import jax.numpy as jnp
import numpy as np
from jax.experimental import pallas as pl
from jax.experimental.pallas import tpu as pltpu

_LB = 256
_LAMBDA_DRIFT = 0.3
_WD, _WX, _WP, _G12, _G31, _WO, _SEQ, _Q, _MEM, _CURD, _MEMD, _F1, _F2 = (
    range(13))


def _fused_body(x_ref, snap_ref, pe_ref, ones_ref,
                wd_ref, wx_ref, wp_ref, g12_ref, g31_ref, wo_ref,
                seqw_ref, qw_ref, memw_ref, curdw_ref, memdw_ref,
                f1_ref, f2_ref,
                bd_ref, bu_ref, bg_ref, bo_ref, seqb_ref, qb_ref, memb_ref,
                curdb_ref, memdb_ref, fuseb_ref,
                y_ref,
                w_scr, snap_bf, x_scr, raw_scr, ms_scr, qd_scr, attn_scr,
                attn_smem, sem, wsem):
    b = pl.program_id(0)
    li = pl.program_id(1)
    T, Lb = snap_ref.shape[1], snap_ref.shape[2]
    D = snap_ref.shape[3]
    nL = pl.num_programs(1) - 1
    L = float(Lb) * nL
    f32 = jnp.float32

    @pl.when(jnp.logical_and(b == 0, li == 0))
    def _load_weights():
        hbm = [wd_ref, wx_ref, wp_ref, g12_ref, g31_ref, wo_ref, seqw_ref,
               qw_ref, memw_ref, curdw_ref, memdw_ref, f1_ref, f2_ref]
        cps = [pltpu.make_async_copy(r, w_scr.at[i], wsem)
               for i, r in enumerate(hbm)]
        for cp in cps:
            cp.start()
        for cp in cps:
            cp.wait()

    @pl.when(li < nL)
    def _phase0():
        snap = snap_ref[0]
        xb = x_ref[0]
        xp = snap[T - 1]
        delta = xb - xp
        df = (jnp.dot(delta, w_scr[_WD], preferred_element_type=f32)
              + bd_ref[...])
        u = (jnp.dot(xb, w_scr[_WX], preferred_element_type=f32)
             - jnp.dot(xp, w_scr[_WP], preferred_element_type=f32)
             + bu_ref[...])
        g = jax.nn.sigmoid(
            jnp.dot(xb, w_scr[_G12], preferred_element_type=f32)
            + jnp.dot(xp, w_scr[_G31], preferred_element_type=f32)
            + bg_ref[...])
        core = g * df + (1.0 - g) * u
        raw = (jnp.dot(core, w_scr[_WO], preferred_element_type=f32)
               + bo_ref[...])
        raw_scr[li] = raw
        x_scr[li] = xb
        for t in range(T):
            snap_bf[li, t] = snap[t].astype(jnp.bfloat16)

        msp = jnp.dot(ones_ref[...], snap.reshape(T * Lb, D),
                      preferred_element_type=f32)
        qp = jnp.sum(xb + raw, axis=0, keepdims=True)
        dp = jnp.sum(delta, axis=0, keepdims=True)
        qdp = jnp.concatenate([qp, dp], axis=0)

        @pl.when(li == 0)
        def _():
            ms_scr[...] = msp
            qd_scr[...] = qdp

        @pl.when(li > 0)
        def _():
            ms_scr[...] += msp
            qd_scr[...] += qdp

    @pl.when(li == nL)
    def _scores_and_phase1():
        ms_mean = ms_scr[...] * (1.0 / L)
        pep = (jnp.dot(pe_ref[...], w_scr[_SEQ],
                       preferred_element_type=f32) + seqb_ref[...])
        m = ms_mean + pep
        memg = (jnp.dot(m, w_scr[_MEM], preferred_element_type=f32)
                + memb_ref[...])
        qg = (jnp.dot(qd_scr[0:1] * (1.0 / L), w_scr[_Q],
                      preferred_element_type=f32) + qb_ref[...])
        content = jnp.sum(qg * memg, axis=-1, keepdims=True) / math.sqrt(D)
        cur = (jnp.dot(qd_scr[1:2] * (1.0 / L), w_scr[_CURD],
                       preferred_element_type=f32) + curdb_ref[...])
        mprev = jnp.concatenate(
            [jnp.zeros((1, D), f32), m[:-1]], axis=0)
        memd = (jnp.dot(m - mprev, w_scr[_MEMD],
                        preferred_element_type=f32) + memdb_ref[...])
        drift = -jnp.mean((cur - memd) ** 2, axis=-1, keepdims=True)
        logits = content + _LAMBDA_DRIFT * drift
        mx = jnp.max(logits, axis=0, keepdims=True)
        e = jnp.exp(logits - mx)
        attn = e / jnp.sum(e, axis=0, keepdims=True)
        attn_scr[...] = attn
        pe_w = jnp.sum(attn * pep, axis=0, keepdims=True)
        pw2 = (jnp.dot(pe_w, w_scr[_F2], preferred_element_type=f32)
               + fuseb_ref[...])
        cp = pltpu.make_async_copy(attn_scr, attn_smem, sem)
        cp.start()
        cp.wait()

        for l in range(nL):
            acc = snap_bf[l, 0].astype(f32) * attn_smem[0, 0]
            for t in range(1, T):
                acc = acc + snap_bf[l, t].astype(f32) * attn_smem[t, 0]
            xb = x_scr[l]
            raw = raw_scr[l]
            enh = acc + pe_w
            fpre = (jnp.dot(xb, w_scr[_F1], preferred_element_type=f32)
                    + jnp.dot(acc, w_scr[_F2], preferred_element_type=f32)
                    + pw2)
            fg = jax.nn.sigmoid(fpre)
            y_ref[0, l * Lb:(l + 1) * Lb] = xb + raw + fg * enh


def _sinusoid_np(T, d):
    half = d // 2
    pos = np.arange(1, T + 1, dtype=np.float64)
    div = np.exp(-math.log(10000.0) * (2.0 * np.arange(half) / d))
    ang = pos[:, None] * div[None, :]
    pe = np.stack([np.sin(ang), np.cos(ang)], axis=-1)
    return jnp.asarray(pe.reshape(T, d), dtype=jnp.float32)


def kernel(x, memory_snapshot, delta_W, delta_b, xproj_W, xproj_b, phys_W,
           phys_b, gate_W, gate_b, outp_W, outp_b, seq_W, seq_b, q_W, q_b,
           mem_W, mem_b, curd_W, curd_b, memd_W, memd_b, fuse_W, fuse_b):
    B, T, L, D = memory_snapshot.shape
    Lb = _LB
    nL = L // Lb

    g1 = gate_W[0:D]
    g12 = g1 + gate_W[D:2 * D]
    g31 = gate_W[2 * D:3 * D] - g1
    f1 = fuse_W[0:D]
    f2 = fuse_W[D:2 * D]
    bu = (xproj_b - phys_b).reshape(1, D)
    r1 = lambda v: v.reshape(1, D)
    pe = _sinusoid_np(T, D)
    ones_bd = jnp.asarray(
        np.kron(np.eye(T, dtype=np.float32), np.ones((1, Lb), np.float32)))

    anyspec = pl.BlockSpec(memory_space=pl.ANY)
    bspec = pl.BlockSpec((1, D), lambda b_, l_: (0, 0))
    y = pl.pallas_call(
        _fused_body,
        grid=(B, nL + 1),
        in_specs=[
            pl.BlockSpec((1, Lb, D),
                         lambda b_, l_: (b_, jnp.minimum(l_, nL - 1), 0)),
            pl.BlockSpec((1, T, Lb, D),
                         lambda b_, l_: (b_, 0, jnp.minimum(l_, nL - 1), 0)),
            pl.BlockSpec((T, D), lambda b_, l_: (0, 0)),
            pl.BlockSpec((T, T * Lb), lambda b_, l_: (0, 0)),
            anyspec, anyspec, anyspec, anyspec, anyspec, anyspec,
            anyspec, anyspec, anyspec, anyspec, anyspec,
            anyspec, anyspec,
            bspec, bspec, bspec, bspec, bspec, bspec, bspec, bspec, bspec,
            bspec,
        ],
        out_specs=pl.BlockSpec((1, L, D), lambda b_, l_: (b_, 0, 0)),
        out_shape=jax.ShapeDtypeStruct((B, L, D), jnp.float32),
        scratch_shapes=[
            pltpu.VMEM((13, D, D), jnp.float32),
            pltpu.VMEM((nL, T, Lb, D), jnp.bfloat16),
            pltpu.VMEM((nL, Lb, D), jnp.float32),
            pltpu.VMEM((nL, Lb, D), jnp.float32),
            pltpu.VMEM((T, D), jnp.float32),
            pltpu.VMEM((2, D), jnp.float32),
            pltpu.VMEM((T, 1), jnp.float32),
            pltpu.SMEM((T, 1), jnp.float32),
            pltpu.SemaphoreType.DMA,
            pltpu.SemaphoreType.DMA,
        ],
        compiler_params=pltpu.CompilerParams(
            dimension_semantics=("parallel", "arbitrary"),
            vmem_limit_bytes=56 * 1024 * 1024,
        ),
        name="dalm_fused",
    )(x, memory_snapshot, pe, ones_bd,
      delta_W, xproj_W, phys_W, g12, g31, outp_W,
      seq_W, q_W, mem_W, curd_W, memd_W, f1, f2,
      r1(delta_b), bu, r1(gate_b), r1(outp_b), r1(seq_b), r1(q_b),
      r1(mem_b), r1(curd_b), r1(memd_b), r1(fuse_b))
    return y
